# Initial kernel scaffold; baseline (speedup 1.0000x reference)
import jax
import jax.numpy as jnp
from jax.experimental import pallas as pl


def kernel(dyn, rel, send, recv, frel_W0, frel_b0, frel_W1, frel_b1, frel_W2, frel_b2, frel_W3, frel_b3, frel_W4, frel_b4, fdyn_W0, fdyn_b0, fdyn_W1, fdyn_b1, fdyn_W2, fdyn_b2, fdyn_W3, fdyn_b3, fdyn_W4, fdyn_b4):
    raise NotImplementedError("write your pallas kernel here")


# speedup vs baseline: 3.0900x; 3.0900x over previous
import functools

import jax
import jax.numpy as jnp
from jax import lax
from jax.experimental import pallas as pl
from jax.experimental.pallas import tpu as pltpu
from jax.experimental.pallas import tpu_sc as plsc

E_BLOCK = 4000
N_BLOCK = 2000

SC_CORES = 2
SC_SUBCORES = 16
NW = SC_CORES * SC_SUBCORES
S_BATCH = 125
S_K = 8
CHUNK = S_BATCH * S_K


def _edge_body(x_ref, w0, b0, w1, b1, w2, b2, w3, b3, w4, b4, out_ref):
    x = x_ref[...]
    x = jnp.maximum(jnp.dot(x, w0[...], preferred_element_type=jnp.float32) + b0[...], 0.0)
    x = jnp.maximum(jnp.dot(x, w1[...], preferred_element_type=jnp.float32) + b1[...], 0.0)
    x = jnp.maximum(jnp.dot(x, w2[...], preferred_element_type=jnp.float32) + b2[...], 0.0)
    x = jnp.maximum(jnp.dot(x, w3[...], preferred_element_type=jnp.float32) + b3[...], 0.0)
    out_ref[...] = jnp.dot(x, w4[...], preferred_element_type=jnp.float32) + b4[...]


def _full_spec(shape):
    return pl.BlockSpec(shape, lambda i: (0,) * len(shape))


def _edge_mlp(x, Ws, bs):
    e, fin = x.shape
    fout = Ws[-1].shape[1]
    specs = [pl.BlockSpec((E_BLOCK, fin), lambda i: (i, 0))]
    args = [x]
    for w, b in zip(Ws, bs):
        specs.append(_full_spec(w.shape))
        args.append(w)
        b2 = b.reshape(1, -1)
        specs.append(_full_spec(b2.shape))
        args.append(b2)
    return pl.pallas_call(
        _edge_body,
        grid=(e // E_BLOCK,),
        in_specs=specs,
        out_specs=pl.BlockSpec((E_BLOCK, fout), lambda i: (i, 0)),
        out_shape=jax.ShapeDtypeStruct((e, fout), jnp.float32),
    )(*args)


def _node_body(dyn_ref, p0_ref, p1_ref, w0a, w0b, b0, w1, b1, w2, b2, w3, b3,
               w4, b4, out_ref):
    h = jnp.dot(dyn_ref[...], w0a[...], preferred_element_type=jnp.float32)
    h = h + jnp.dot(p0_ref[...] + p1_ref[...], w0b[...],
                    preferred_element_type=jnp.float32)
    h = jnp.maximum(h + b0[...], 0.0)
    h = jnp.maximum(jnp.dot(h, w1[...], preferred_element_type=jnp.float32) + b1[...], 0.0)
    h = jnp.maximum(jnp.dot(h, w2[...], preferred_element_type=jnp.float32) + b2[...], 0.0)
    h = jnp.maximum(jnp.dot(h, w3[...], preferred_element_type=jnp.float32) + b3[...], 0.0)
    out_ref[...] = jnp.dot(h, w4[...], preferred_element_type=jnp.float32) + b4[...]


def _node_mlp(dyn, p0, p1, Ws, bs):
    n, fdyn = dyn.shape
    fagg = p0.shape[1]
    fout = Ws[-1].shape[1]
    w0a = Ws[0][:fdyn]
    w0b = Ws[0][fdyn:]
    specs = [
        pl.BlockSpec((N_BLOCK, fdyn), lambda i: (i, 0)),
        pl.BlockSpec((N_BLOCK, fagg), lambda i: (i, 0)),
        pl.BlockSpec((N_BLOCK, fagg), lambda i: (i, 0)),
        _full_spec(w0a.shape),
        _full_spec(w0b.shape),
    ]
    args = [dyn, p0, p1, w0a, w0b]
    b2 = bs[0].reshape(1, -1)
    specs.append(_full_spec(b2.shape))
    args.append(b2)
    for w, b in zip(Ws[1:], bs[1:]):
        specs.append(_full_spec(w.shape))
        args.append(w)
        b2 = b.reshape(1, -1)
        specs.append(_full_spec(b2.shape))
        args.append(b2)
    return pl.pallas_call(
        _node_body,
        grid=(n // N_BLOCK,),
        in_specs=specs,
        out_specs=pl.BlockSpec((N_BLOCK, fout), lambda i: (i, 0)),
        out_shape=jax.ShapeDtypeStruct((n, fout), jnp.float32),
    )(*args)


def _sc_scatter(msg, recv2, n):
    e = msg.shape[0]
    rounds = e // (NW * CHUNK)
    rows_per_worker = rounds * S_K
    align = 8 * SC_SUBCORES
    n_pad = ((n + align - 1) // align) * align
    stripe = n_pad // SC_SUBCORES

    zstripe = jnp.zeros((stripe, 16), jnp.float32)
    mesh = plsc.VectorSubcoreMesh(core_axis_name="c", subcore_axis_name="s")

    @functools.partial(
        pl.kernel,
        mesh=mesh,
        out_type=jax.ShapeDtypeStruct((SC_CORES, n_pad, 16), jnp.float32),
        scratch_types=[
            pltpu.VMEM((S_K, S_BATCH), jnp.int32),
            pltpu.VMEM((CHUNK, 16), jnp.float32),
            pltpu.VMEM_SHARED((n_pad, 16), jnp.float32),
        ],
        compiler_params=pltpu.CompilerParams(use_tc_tiling_on_sc=False),
    )
    def k(msg_hbm, recv_hbm, z_hbm, out_hbm, idx_v, msg_v, agg_sh):
        c = lax.axis_index("c")
        s = lax.axis_index("s")
        w = s * SC_CORES + c

        pltpu.sync_copy(z_hbm, agg_sh.at[pl.ds(s * stripe, stripe)])
        plsc.subcore_barrier()

        @pl.loop(0, rounds)
        def _(r):
            row0 = w * rows_per_worker + r * S_K
            pltpu.sync_copy(recv_hbm.at[pl.ds(row0, S_K)], idx_v)
            pltpu.sync_copy(msg_hbm.at[pl.ds(row0 * S_BATCH, CHUNK)], msg_v)
            for j in range(S_K):
                pltpu.sync_copy(
                    msg_v.at[pl.ds(j * S_BATCH, S_BATCH)],
                    agg_sh.at[idx_v.at[j]],
                    add=True,
                )

        plsc.subcore_barrier()
        pltpu.sync_copy(agg_sh.at[pl.ds(s * stripe, stripe)],
                        out_hbm.at[c, pl.ds(s * stripe, stripe)])

    return k(msg, recv2, zstripe)[:, :n, :]


def kernel(dyn, rel, send, recv,
           frel_W0, frel_b0, frel_W1, frel_b1, frel_W2, frel_b2, frel_W3,
           frel_b3, frel_W4, frel_b4,
           fdyn_W0, fdyn_b0, fdyn_W1, fdyn_b1, fdyn_W2, fdyn_b2, fdyn_W3,
           fdyn_b3, fdyn_W4, fdyn_b4):
    frel_Ws = [frel_W0, frel_W1, frel_W2, frel_W3, frel_W4]
    frel_bs = [frel_b0, frel_b1, frel_b2, frel_b3, frel_b4]
    fdyn_Ws = [fdyn_W0, fdyn_W1, fdyn_W2, fdyn_W3, fdyn_W4]
    fdyn_bs = [fdyn_b0, fdyn_b1, fdyn_b2, fdyn_b3, fdyn_b4]
    b, n, _ = dyn.shape
    e = rel.shape[1]

    msg = _edge_mlp(rel.reshape(e, rel.shape[-1]), frel_Ws, frel_bs)

    recv2 = recv.reshape(e // S_BATCH, S_BATCH)
    parts = _sc_scatter(msg, recv2, n)

    out = _node_mlp(dyn.reshape(n, dyn.shape[-1]), parts[0], parts[1],
                    fdyn_Ws, fdyn_bs)
    return out.reshape(b, n, out.shape[-1])
